# Initial kernel scaffold; baseline (speedup 1.0000x reference)
import jax
import jax.numpy as jnp
from jax.experimental import pallas as pl


def kernel(image_features, text_features_all):
    raise NotImplementedError("write your pallas kernel here")


# speedup vs baseline: 2.8457x; 2.8457x over previous
import functools

import jax
import jax.numpy as jnp
from jax.experimental import pallas as pl
from jax.experimental.pallas import tpu as pltpu

_T = 0.07
_INV_T = 1.0 / _T
_LOG2E = 1.4426950408889634


def _main_body(img_ref, txt_ref, row_ref, col_ref, diag_ref, *,
               ib, jb, ch, v, half):
    p = pl.program_id(0)
    j = pl.program_id(1)
    i = pl.program_id(2)

    @pl.when(jnp.logical_and(j == 0, i == 0))
    def _init_diag():
        diag_ref[...] = jnp.zeros_like(diag_ref)

    im = img_ref[...]
    rs_total = jnp.zeros((ib,), jnp.float32)
    c1 = _LOG2E * _INV_T

    for c in range(jb // ch):
        tc = txt_ref[c * ch:(c + 1) * ch, :]
        s = jax.lax.dot_general(
            im, tc, (((1,), (1,)), ((), ())),
            preferred_element_type=jnp.float32)
        e = jnp.exp2((s - 1.0) * c1)
        rs_total = rs_total + jnp.sum(e, axis=1)
        cs = jnp.sum(e, axis=0)
        off = j * jb + c * ch
        cur = col_ref[0, 0, pl.ds(off, ch)]
        col_ref[0, 0, pl.ds(off, ch)] = jnp.where(i > 0, cur, 0.0) + cs

        g0 = p * half + off
        row0 = i * ib
        overlap = jnp.logical_and(g0 + ch > v * row0,
                                  g0 < v * (row0 + ib))

        @pl.when(overlap)
        def _diag():
            sl = s * _INV_T
            ii = jax.lax.broadcasted_iota(jnp.int32, (ib, ch), 0)
            jj = jax.lax.broadcasted_iota(jnp.int32, (ib, ch), 1)
            t = (jj + (g0 - v * row0)) - v * ii
            msk = t.astype(jnp.uint32) < v
            dsum = jnp.sum(jnp.where(msk, sl, 0.0), axis=0)
            d128 = (dsum[0:128] + dsum[128:256]
                    + dsum[256:384] + dsum[384:512])
            diag_ref[0, 0, :] = diag_ref[0, 0, :] + d128

    cur_r = row_ref[0, 0, pl.ds(i * ib, ib)]
    row_ref[0, 0, pl.ds(i * ib, ib)] = jnp.where(j > 0, cur_r, 0.0) + rs_total


def _fin_body(row_ref, col_ref, diag_ref, out_ref, *, b, v):
    r = row_ref[0, 0, :] + row_ref[1, 0, :]
    lr = jnp.sum(jnp.log(r))
    lc = jnp.sum(jnp.log(col_ref[...]))
    dg = jnp.sum(diag_ref[...])
    bv = b * v
    loss = 0.5 * (lr / b + lc / bv + 2.0 * _INV_T - 2.0 * dg / bv)
    out_ref[...] = loss[None, None]


@jax.jit
def kernel(image_features, text_features_all):
    b, v, d = text_features_all.shape
    bv = b * v
    ib, jb, ch = 512, 2048, 512
    half = bv // 2
    n_j = half // jb
    n_i = b // ib

    img_bf = image_features.astype(jnp.bfloat16)
    txt_bf = text_features_all.reshape(bv, d).astype(jnp.bfloat16)

    row_p, col_p, diag_p = pl.pallas_call(
        functools.partial(_main_body, ib=ib, jb=jb, ch=ch, v=v, half=half),
        grid=(2, n_j, n_i),
        in_specs=[
            pl.BlockSpec((ib, d), lambda p, j, i: (i, 0)),
            pl.BlockSpec((jb, d), lambda p, j, i, nj=n_j: (p * nj + j, 0)),
        ],
        out_specs=[
            pl.BlockSpec((1, 1, b), lambda p, j, i: (p, 0, 0)),
            pl.BlockSpec((1, 1, half), lambda p, j, i: (p, 0, 0)),
            pl.BlockSpec((1, 1, 128), lambda p, j, i: (p, 0, 0)),
        ],
        out_shape=[
            jax.ShapeDtypeStruct((2, 1, b), jnp.float32),
            jax.ShapeDtypeStruct((2, 1, half), jnp.float32),
            jax.ShapeDtypeStruct((2, 1, 128), jnp.float32),
        ],
        compiler_params=pltpu.CompilerParams(
            dimension_semantics=("parallel", "arbitrary", "arbitrary")),
    )(img_bf, txt_bf)

    out = pl.pallas_call(
        functools.partial(_fin_body, b=b, v=v),
        out_shape=jax.ShapeDtypeStruct((1, 1), jnp.float32),
    )(row_p, col_p, diag_p)

    return out[0, 0]
